# Initial kernel scaffold; baseline (speedup 1.0000x reference)
import functools

import jax
import jax.numpy as jnp
from jax import lax
from jax.experimental import pallas as pl
from jax.experimental.pallas import tpu as pltpu
from jax.experimental.pallas import tpu_sc as plsc

_N = 10000
_E = 320000
_D = 128
_NTILE = 16
_PT = _E // _NTILE
_CH = 128
_NCH = _PT // _CH
_TAIL = _PT - _NCH * _CH
_RPT = _N // _NTILE
_DCH = 1000
_NDT = _N // _DCH


def _sc_body(with_deg, *refs):
    if with_deg:
        (h, s0m, s0t, d0m, d0t, s1m, s1t, d1m, d1t, z2d, z1d,
         out0, out1, deg0, deg1,
         src2d, dst2d, srct, dstt, rows, rowst, acc, gsem, ones, dacc) = refs
    else:
        (h, s0m, s0t, d0m, d0t, s1m, s1t, d1m, d1t, z2d, z1d,
         out0, out1,
         src2d, dst2d, srct, dstt, rows, rowst, acc, gsem) = refs
        ones = dacc = deg0 = deg1 = None

    c = lax.axis_index("c")
    t = lax.axis_index("s")

    pltpu.sync_copy(z2d, acc.at[pl.ds(t * _RPT, _RPT)])
    if with_deg:
        @pl.when(t < _NDT)
        def _():
            pltpu.sync_copy(z1d, dacc.at[pl.ds(t * _DCH, _DCH)])
        for i in range(_CH // 16):
            ones[pl.ds(i * 16, 16)] = jnp.ones((16,), jnp.float32)

    @pl.when(c == 0)
    def _():
        pltpu.sync_copy(s0m.at[t], src2d)
        pltpu.sync_copy(d0m.at[t], dst2d)
        pltpu.sync_copy(s0t.at[t], srct)
        pltpu.sync_copy(d0t.at[t], dstt)

    @pl.when(c == 1)
    def _():
        pltpu.sync_copy(s1m.at[t], src2d)
        pltpu.sync_copy(d1m.at[t], dst2d)
        pltpu.sync_copy(s1t.at[t], srct)
        pltpu.sync_copy(d1t.at[t], dstt)

    plsc.subcore_barrier()

    @pl.loop(0, _NCH)
    def _(j):
        pltpu.async_copy(h.at[src2d.at[j]], rows, gsem).wait()
        pltpu.sync_copy(rows, acc.at[dst2d.at[j]], add=True)
        if with_deg:
            pltpu.sync_copy(ones, dacc.at[dst2d.at[j]], add=True)

    pltpu.async_copy(h.at[srct.at[0]], rowst, gsem).wait()
    pltpu.sync_copy(rowst, acc.at[dstt.at[0]], add=True)
    if with_deg:
        pltpu.sync_copy(ones.at[pl.ds(0, _TAIL)], dacc.at[dstt.at[0]], add=True)

    plsc.subcore_barrier()

    @pl.when(c == 0)
    def _():
        pltpu.sync_copy(acc.at[pl.ds(t * _RPT, _RPT)],
                        out0.at[pl.ds(t * _RPT, _RPT)])
        if with_deg:
            @pl.when(t < _NDT)
            def _():
                pltpu.sync_copy(dacc.at[pl.ds(t * _DCH, _DCH)],
                                deg0.at[pl.ds(t * _DCH, _DCH)])

    @pl.when(c == 1)
    def _():
        pltpu.sync_copy(acc.at[pl.ds(t * _RPT, _RPT)],
                        out1.at[pl.ds(t * _RPT, _RPT)])
        if with_deg:
            @pl.when(t < _NDT)
            def _():
                pltpu.sync_copy(dacc.at[pl.ds(t * _DCH, _DCH)],
                                deg1.at[pl.ds(t * _DCH, _DCH)])


def _make_sc(with_deg):
    out_type = [jax.ShapeDtypeStruct((_N, _D), jnp.float32),
                jax.ShapeDtypeStruct((_N, _D), jnp.float32)]
    if with_deg:
        out_type += [jax.ShapeDtypeStruct((_N,), jnp.float32),
                     jax.ShapeDtypeStruct((_N,), jnp.float32)]
    scratch = [
        pltpu.VMEM((_NCH, _CH), jnp.int32),
        pltpu.VMEM((_NCH, _CH), jnp.int32),
        pltpu.VMEM((1, _TAIL), jnp.int32),
        pltpu.VMEM((1, _TAIL), jnp.int32),
        pltpu.VMEM((_CH, _D), jnp.float32),
        pltpu.VMEM((_TAIL, _D), jnp.float32),
        pltpu.VMEM_SHARED((_N, _D), jnp.float32),
        pltpu.SemaphoreType.DMA,
    ]
    if with_deg:
        scratch += [
            pltpu.VMEM((_CH,), jnp.float32),
            pltpu.VMEM_SHARED((_N,), jnp.float32),
        ]
    mesh = plsc.VectorSubcoreMesh(core_axis_name="c", subcore_axis_name="s")
    return pl.kernel(functools.partial(_sc_body, with_deg),
                     out_type=tuple(out_type), mesh=mesh,
                     scratch_types=scratch)


_BLK = 2000
_NB = _N // _BLK


def _tc_body(relu, h, s0, s1, d0, d1, ws0, wn0, b0, ws1, wn1, b1, g, bt,
             out, acc, sums):
    ph = pl.program_id(0)
    j = pl.program_id(1)

    @pl.when(ph == 0)
    def _():
        hn0 = s0[...] / jnp.maximum(d0[...], 1.0)
        hn1 = s1[...] / jnp.maximum(d1[...], 1.0)
        o0 = (jnp.dot(h[...], ws0[...], preferred_element_type=jnp.float32)
              + jnp.dot(hn0, wn0[...], preferred_element_type=jnp.float32)
              + b0[...])
        o1 = (jnp.dot(h[...], ws1[...], preferred_element_type=jnp.float32)
              + jnp.dot(hn1, wn1[...], preferred_element_type=jnp.float32)
              + b1[...])
        if relu:
            o0 = jnp.maximum(o0, 0.0)
            o1 = jnp.maximum(o1, 0.0)
        a = o0 + o1
        acc[pl.ds(j * _BLK, _BLK), :] = a
        cs = jnp.sum(a, axis=0, keepdims=True)
        cq = jnp.sum(a * a, axis=0, keepdims=True)

        @pl.when(j == 0)
        def _():
            sums[0:1, :] = cs
            sums[1:2, :] = cq

        @pl.when(j > 0)
        def _():
            sums[0:1, :] = sums[0:1, :] + cs
            sums[1:2, :] = sums[1:2, :] + cq

    @pl.when(ph == 1)
    def _():
        mean = sums[0:1, :] * (1.0 / _N)
        var = sums[1:2, :] * (1.0 / _N) - mean * mean
        a = acc[pl.ds(j * _BLK, _BLK), :]
        out[...] = (a - mean) * lax.rsqrt(var + 1e-5) * g[...] + bt[...]


def _make_tc(relu):
    blk = lambda p, j: (j, 0)
    whole = lambda p, j: (0, 0)
    in_specs = [
        pl.BlockSpec((_BLK, _D), blk),
        pl.BlockSpec((_BLK, _D), blk),
        pl.BlockSpec((_BLK, _D), blk),
        pl.BlockSpec((_BLK, 1), blk),
        pl.BlockSpec((_BLK, 1), blk),
        pl.BlockSpec((_D, _D), whole),
        pl.BlockSpec((_D, _D), whole),
        pl.BlockSpec((1, _D), whole),
        pl.BlockSpec((_D, _D), whole),
        pl.BlockSpec((_D, _D), whole),
        pl.BlockSpec((1, _D), whole),
        pl.BlockSpec((1, _D), whole),
        pl.BlockSpec((1, _D), whole),
    ]
    return pl.pallas_call(
        functools.partial(_tc_body, relu),
        grid=(2, _NB),
        in_specs=in_specs,
        out_specs=pl.BlockSpec((_BLK, _D), blk),
        out_shape=jax.ShapeDtypeStruct((_N, _D), jnp.float32),
        scratch_shapes=[
            pltpu.VMEM((_N, _D), jnp.float32),
            pltpu.VMEM((8, _D), jnp.float32),
        ],
    )


def _edge_layout(ei):
    s = ei[0].reshape(_NTILE, _PT)
    d = ei[1].reshape(_NTILE, _PT)
    sm = s[:, :_NCH * _CH].reshape(_NTILE, _NCH, _CH)
    st = s[:, _NCH * _CH:].reshape(_NTILE, 1, _TAIL)
    dm = d[:, :_NCH * _CH].reshape(_NTILE, _NCH, _CH)
    dt = d[:, _NCH * _CH:].reshape(_NTILE, 1, _TAIL)
    return sm, st, dm, dt


def kernel(x, edge_index_0, edge_index_1,
           W_self_0_0, W_neigh_0_0, b_0_0,
           W_self_0_1, W_neigh_0_1, b_0_1,
           gamma_0, beta_0,
           W_self_1_0, W_neigh_1_0, b_1_0,
           W_self_1_1, W_neigh_1_1, b_1_1,
           gamma_1, beta_1,
           W_self_2_0, W_neigh_2_0, b_2_0,
           W_self_2_1, W_neigh_2_1, b_2_1,
           gamma_2, beta_2):
    s0m, s0t, d0m, d0t = _edge_layout(edge_index_0)
    s1m, s1t, d1m, d1t = _edge_layout(edge_index_1)
    z2d = jnp.zeros((_RPT, _D), jnp.float32)
    z1d = jnp.zeros((_DCH,), jnp.float32)

    sc_first = _make_sc(True)
    sc_rest = _make_sc(False)
    tc_mid = _make_tc(True)
    tc_last = _make_tc(False)

    edge_args = (s0m, s0t, d0m, d0t, s1m, s1t, d1m, d1t, z2d, z1d)

    layer_ws = [
        (W_self_0_0, W_neigh_0_0, b_0_0, W_self_0_1, W_neigh_0_1, b_0_1,
         gamma_0, beta_0),
        (W_self_1_0, W_neigh_1_0, b_1_0, W_self_1_1, W_neigh_1_1, b_1_1,
         gamma_1, beta_1),
        (W_self_2_0, W_neigh_2_0, b_2_0, W_self_2_1, W_neigh_2_1, b_2_1,
         gamma_2, beta_2),
    ]

    h = x
    deg0_c = deg1_c = None
    for l in range(3):
        if l == 0:
            S0, S1, deg0, deg1 = sc_first(h, *edge_args)
            deg0_c = deg0.reshape(_N, 1)
            deg1_c = deg1.reshape(_N, 1)
        else:
            S0, S1 = sc_rest(h, *edge_args)
        ws0, wn0, b0, ws1, wn1, b1, g, bt = layer_ws[l]
        tc = tc_mid if l < 2 else tc_last
        h = tc(h, S0, S1, deg0_c, deg1_c,
               ws0, wn0, b0.reshape(1, _D), ws1, wn1, b1.reshape(1, _D),
               g.reshape(1, _D), bt.reshape(1, _D))
    return h


# speedup vs baseline: 2.8710x; 2.8710x over previous
import functools

import jax
import jax.numpy as jnp
from jax import lax
from jax.experimental import pallas as pl
from jax.experimental.pallas import tpu as pltpu
from jax.experimental.pallas import tpu_sc as plsc

_N = 10000
_E = 320000
_D = 128
_NTILE = 16
_PT = _E // _NTILE
_CH = 128
_NCHP = 160
_PTP = _NCHP * _CH
_G = 16
_NSUP = _NCHP // _G
_NACC = 10048
_RPT = 624
_RPT_LAST = _N - 15 * _RPT
_ZLAST = _NACC - 15 * _RPT
_DCH = 1000
_NDT = _N // _DCH


def _sc_body(with_deg, *refs):
    if with_deg:
        (h, s0, d0, s1, d1, z2d, z1d,
         out, deg0, deg1,
         sbuf, dbuf, rows, acc, gsem, ones, dacc, dstage) = refs
    else:
        (h, s0, d0, s1, d1, z2d, z1d,
         out,
         sbuf, dbuf, rows, acc, gsem) = refs
        ones = dacc = dstage = deg0 = deg1 = None

    c = lax.axis_index("c")
    t = lax.axis_index("s")

    @pl.when(t < _NTILE - 1)
    def _():
        pltpu.sync_copy(z2d.at[pl.ds(0, _RPT)], acc.at[pl.ds(t * _RPT, _RPT)])

    @pl.when(t == _NTILE - 1)
    def _():
        pltpu.sync_copy(z2d, acc.at[pl.ds((_NTILE - 1) * _RPT, _ZLAST)])

    if with_deg:
        @pl.when(t < _NDT)
        def _():
            pltpu.sync_copy(z1d, dstage)
            pltpu.sync_copy(dstage, dacc.at[pl.ds(t * _DCH, _DCH)])
        for i in range(_CH // 16):
            ones[pl.ds(i * 16, 16)] = jnp.ones((16,), jnp.float32)

    plsc.subcore_barrier()

    def run(sm, dm):
        @pl.loop(0, _NSUP)
        def _(g):
            pltpu.sync_copy(sm.at[t, pl.ds(g * _G, _G)], sbuf)
            pltpu.sync_copy(dm.at[t, pl.ds(g * _G, _G)], dbuf)

            @pl.loop(0, _G)
            def _(j):
                pltpu.async_copy(h.at[sbuf.at[j]], rows, gsem).wait()
                pltpu.sync_copy(rows, acc.at[dbuf.at[j]], add=True)
                if with_deg:
                    pltpu.sync_copy(ones, dacc.at[dbuf.at[j]], add=True)

    @pl.when(c == 0)
    def _():
        run(s0, d0)

    @pl.when(c == 1)
    def _():
        run(s1, d1)

    plsc.subcore_barrier()

    @pl.when(t < _NTILE - 1)
    def _():
        pltpu.sync_copy(acc.at[pl.ds(t * _RPT, _RPT)],
                        out.at[c, pl.ds(t * _RPT, _RPT)])

    @pl.when(t == _NTILE - 1)
    def _():
        pltpu.sync_copy(acc.at[pl.ds((_NTILE - 1) * _RPT, _RPT_LAST)],
                        out.at[c, pl.ds((_NTILE - 1) * _RPT, _RPT_LAST)])

    if with_deg:
        @pl.when(t < _NDT)
        def _():
            pltpu.sync_copy(dacc.at[pl.ds(t * _DCH, _DCH)], dstage)

            @pl.when(c == 0)
            def _():
                pltpu.sync_copy(dstage, deg0.at[pl.ds(t * _DCH, _DCH)])

            @pl.when(c == 1)
            def _():
                pltpu.sync_copy(dstage, deg1.at[pl.ds(t * _DCH, _DCH)])


def _make_sc(with_deg):
    out_type = [jax.ShapeDtypeStruct((2, _N, _D), jnp.float32)]
    if with_deg:
        out_type += [jax.ShapeDtypeStruct((_N,), jnp.float32),
                     jax.ShapeDtypeStruct((_N,), jnp.float32)]
    scratch = [
        pltpu.VMEM((_G, _CH), jnp.int32),
        pltpu.VMEM((_G, _CH), jnp.int32),
        pltpu.VMEM((_CH, _D), jnp.float32),
        pltpu.VMEM_SHARED((_NACC, _D), jnp.float32),
        pltpu.SemaphoreType.DMA,
    ]
    if with_deg:
        scratch += [
            pltpu.VMEM((_CH,), jnp.float32),
            pltpu.VMEM_SHARED((_NACC,), jnp.float32),
            pltpu.VMEM((_DCH,), jnp.float32),
        ]
    mesh = plsc.VectorSubcoreMesh(core_axis_name="c", subcore_axis_name="s")
    return pl.kernel(functools.partial(_sc_body, with_deg),
                     out_type=tuple(out_type), mesh=mesh,
                     scratch_types=scratch)


_BLK = 2000
_NB = _N // _BLK


def _tc_body(relu, h, s0, s1, d0, d1, ws0, wn0, b0, ws1, wn1, b1, g, bt,
             out, acc_s, sums):
    ph = pl.program_id(0)
    j = pl.program_id(1)

    @pl.when(ph == 0)
    def _():
        hn0 = s0[0] / jnp.maximum(d0[0], 1.0)
        hn1 = s1[0] / jnp.maximum(d1[0], 1.0)
        o0 = (jnp.dot(h[...], ws0[...], preferred_element_type=jnp.float32)
              + jnp.dot(hn0, wn0[...], preferred_element_type=jnp.float32)
              + b0[...])
        o1 = (jnp.dot(h[...], ws1[...], preferred_element_type=jnp.float32)
              + jnp.dot(hn1, wn1[...], preferred_element_type=jnp.float32)
              + b1[...])
        if relu:
            o0 = jnp.maximum(o0, 0.0)
            o1 = jnp.maximum(o1, 0.0)
        a = o0 + o1
        acc_s[pl.ds(j * _BLK, _BLK), :] = a
        cs = jnp.sum(a, axis=0, keepdims=True)
        cq = jnp.sum(a * a, axis=0, keepdims=True)

        @pl.when(j == 0)
        def _():
            sums[0:1, :] = cs
            sums[1:2, :] = cq

        @pl.when(j > 0)
        def _():
            sums[0:1, :] = sums[0:1, :] + cs
            sums[1:2, :] = sums[1:2, :] + cq

    @pl.when(ph == 1)
    def _():
        mean = sums[0:1, :] * (1.0 / _N)
        var = sums[1:2, :] * (1.0 / _N) - mean * mean
        a = acc_s[pl.ds(j * _BLK, _BLK), :]
        out[...] = (a - mean) * lax.rsqrt(var + 1e-5) * g[...] + bt[...]


def _make_tc(relu):
    blk = lambda p, j: (j, 0)
    sblk = lambda p, j: (0, j, 0)
    dblk = lambda p, j: (0, j, 0)
    whole = lambda p, j: (0, 0)
    in_specs = [
        pl.BlockSpec((_BLK, _D), blk),
        pl.BlockSpec((1, _BLK, _D), sblk),
        pl.BlockSpec((1, _BLK, _D), lambda p, j: (1, j, 0)),
        pl.BlockSpec((1, _BLK, 1), dblk),
        pl.BlockSpec((1, _BLK, 1), lambda p, j: (1, j, 0)),
        pl.BlockSpec((_D, _D), whole),
        pl.BlockSpec((_D, _D), whole),
        pl.BlockSpec((1, _D), whole),
        pl.BlockSpec((_D, _D), whole),
        pl.BlockSpec((_D, _D), whole),
        pl.BlockSpec((1, _D), whole),
        pl.BlockSpec((1, _D), whole),
        pl.BlockSpec((1, _D), whole),
    ]
    return pl.pallas_call(
        functools.partial(_tc_body, relu),
        grid=(2, _NB),
        in_specs=in_specs,
        out_specs=pl.BlockSpec((_BLK, _D), blk),
        out_shape=jax.ShapeDtypeStruct((_N, _D), jnp.float32),
        scratch_shapes=[
            pltpu.VMEM((_N, _D), jnp.float32),
            pltpu.VMEM((8, _D), jnp.float32),
        ],
    )


def _edge_layout(ei):
    s = ei[0].reshape(_NTILE, _PT)
    d = ei[1].reshape(_NTILE, _PT)
    s = jnp.pad(s, ((0, 0), (0, _PTP - _PT)))
    d = jnp.pad(d, ((0, 0), (0, _PTP - _PT)), constant_values=_N)
    return s.reshape(_NTILE, _NCHP, _CH), d.reshape(_NTILE, _NCHP, _CH)


def kernel(x, edge_index_0, edge_index_1,
           W_self_0_0, W_neigh_0_0, b_0_0,
           W_self_0_1, W_neigh_0_1, b_0_1,
           gamma_0, beta_0,
           W_self_1_0, W_neigh_1_0, b_1_0,
           W_self_1_1, W_neigh_1_1, b_1_1,
           gamma_1, beta_1,
           W_self_2_0, W_neigh_2_0, b_2_0,
           W_self_2_1, W_neigh_2_1, b_2_1,
           gamma_2, beta_2):
    s0, d0 = _edge_layout(edge_index_0)
    s1, d1 = _edge_layout(edge_index_1)
    z2d = jnp.zeros((_ZLAST, _D), jnp.float32)
    z1d = jnp.zeros((_DCH,), jnp.float32)

    sc_first = _make_sc(True)
    sc_rest = _make_sc(False)
    tc_mid = _make_tc(True)
    tc_last = _make_tc(False)

    edge_args = (s0, d0, s1, d1, z2d, z1d)

    layer_ws = [
        (W_self_0_0, W_neigh_0_0, b_0_0, W_self_0_1, W_neigh_0_1, b_0_1,
         gamma_0, beta_0),
        (W_self_1_0, W_neigh_1_0, b_1_0, W_self_1_1, W_neigh_1_1, b_1_1,
         gamma_1, beta_1),
        (W_self_2_0, W_neigh_2_0, b_2_0, W_self_2_1, W_neigh_2_1, b_2_1,
         gamma_2, beta_2),
    ]

    h = x
    deg = None
    for l in range(3):
        if l == 0:
            S, g0, g1 = sc_first(h, *edge_args)
            deg = jnp.stack([g0, g1]).reshape(2, _N, 1)
        else:
            (S,) = sc_rest(h, *edge_args)
        ws0, wn0, b0, ws1, wn1, b1, g, bt = layer_ws[l]
        tc = tc_mid if l < 2 else tc_last
        h = tc(h, S, S, deg, deg,
               ws0, wn0, b0.reshape(1, _D), ws1, wn1, b1.reshape(1, _D),
               g.reshape(1, _D), bt.reshape(1, _D))
    return h


# speedup vs baseline: 3.1527x; 1.0981x over previous
import functools

import jax
import jax.numpy as jnp
from jax import lax
from jax.experimental import pallas as pl
from jax.experimental.pallas import tpu as pltpu
from jax.experimental.pallas import tpu_sc as plsc

_N = 10000
_E = 320000
_D = 128
_NTILE = 16
_PT = _E // _NTILE
_CH = 128
_NCHP = 160
_PTP = _NCHP * _CH
_G = 16
_NSUP = _NCHP // _G
_NACC = 10048
_RPT = 624
_RPT_LAST = _N - 15 * _RPT
_ZLAST = _NACC - 15 * _RPT
_DCH = 1000
_NDT = _N // _DCH


def _sc_body(with_deg, *refs):
    if with_deg:
        (h, s0, d0, s1, d1, z2d, z1d,
         out, deg0, deg1,
         sbuf, dbuf, rows0, rows1, acc, gsem0, gsem1,
         ones, dacc, dstage) = refs
    else:
        (h, s0, d0, s1, d1, z2d, z1d,
         out,
         sbuf, dbuf, rows0, rows1, acc, gsem0, gsem1) = refs
        ones = dacc = dstage = deg0 = deg1 = None
    rowbuf = (rows0, rows1)
    gsem = (gsem0, gsem1)

    c = lax.axis_index("c")
    t = lax.axis_index("s")

    @pl.when(t < _NTILE - 1)
    def _():
        pltpu.sync_copy(z2d.at[pl.ds(0, _RPT)], acc.at[pl.ds(t * _RPT, _RPT)])

    @pl.when(t == _NTILE - 1)
    def _():
        pltpu.sync_copy(z2d, acc.at[pl.ds((_NTILE - 1) * _RPT, _ZLAST)])

    if with_deg:
        @pl.when(t < _NDT)
        def _():
            pltpu.sync_copy(z1d, dstage)
            pltpu.sync_copy(dstage, dacc.at[pl.ds(t * _DCH, _DCH)])
        for i in range(_CH // 16):
            ones[pl.ds(i * 16, 16)] = jnp.ones((16,), jnp.float32)

    plsc.subcore_barrier()

    def run(sm, dm):
        @pl.loop(0, _NSUP)
        def _(g):
            pltpu.sync_copy(sm.at[t, pl.ds(g * _G, _G)], sbuf)
            pltpu.sync_copy(dm.at[t, pl.ds(g * _G, _G)], dbuf)

            copies = [None, None]
            copies[0] = pltpu.async_copy(h.at[sbuf.at[0]], rowbuf[0],
                                         gsem[0])
            for j in range(_G):
                b = j % 2
                copies[b].wait()
                if j + 1 < _G:
                    nb = (j + 1) % 2
                    copies[nb] = pltpu.async_copy(h.at[sbuf.at[j + 1]],
                                                  rowbuf[nb], gsem[nb])
                pltpu.sync_copy(rowbuf[b], acc.at[dbuf.at[j]], add=True)
                if with_deg:
                    pltpu.sync_copy(ones, dacc.at[dbuf.at[j]], add=True)

    @pl.when(c == 0)
    def _():
        run(s0, d0)

    @pl.when(c == 1)
    def _():
        run(s1, d1)

    plsc.subcore_barrier()

    @pl.when(t < _NTILE - 1)
    def _():
        pltpu.sync_copy(acc.at[pl.ds(t * _RPT, _RPT)],
                        out.at[c, pl.ds(t * _RPT, _RPT)])

    @pl.when(t == _NTILE - 1)
    def _():
        pltpu.sync_copy(acc.at[pl.ds((_NTILE - 1) * _RPT, _RPT_LAST)],
                        out.at[c, pl.ds((_NTILE - 1) * _RPT, _RPT_LAST)])

    if with_deg:
        @pl.when(t < _NDT)
        def _():
            pltpu.sync_copy(dacc.at[pl.ds(t * _DCH, _DCH)], dstage)

            @pl.when(c == 0)
            def _():
                pltpu.sync_copy(dstage, deg0.at[pl.ds(t * _DCH, _DCH)])

            @pl.when(c == 1)
            def _():
                pltpu.sync_copy(dstage, deg1.at[pl.ds(t * _DCH, _DCH)])


def _make_sc(with_deg):
    out_type = [jax.ShapeDtypeStruct((2, _N, _D), jnp.float32)]
    if with_deg:
        out_type += [jax.ShapeDtypeStruct((_N,), jnp.float32),
                     jax.ShapeDtypeStruct((_N,), jnp.float32)]
    scratch = [
        pltpu.VMEM((_G, _CH), jnp.int32),
        pltpu.VMEM((_G, _CH), jnp.int32),
        pltpu.VMEM((_CH, _D), jnp.float32),
        pltpu.VMEM((_CH, _D), jnp.float32),
        pltpu.VMEM_SHARED((_NACC, _D), jnp.float32),
        pltpu.SemaphoreType.DMA,
        pltpu.SemaphoreType.DMA,
    ]
    if with_deg:
        scratch += [
            pltpu.VMEM((_CH,), jnp.float32),
            pltpu.VMEM_SHARED((_NACC,), jnp.float32),
            pltpu.VMEM((_DCH,), jnp.float32),
        ]
    mesh = plsc.VectorSubcoreMesh(core_axis_name="c", subcore_axis_name="s")
    return pl.kernel(functools.partial(_sc_body, with_deg),
                     out_type=tuple(out_type), mesh=mesh,
                     scratch_types=scratch)


_BLK = 2000
_NB = _N // _BLK


def _tc_body(relu, h, s0, s1, d0, d1, ws0, wn0, b0, ws1, wn1, b1, g, bt,
             out, acc_s, sums):
    ph = pl.program_id(0)
    j = pl.program_id(1)

    @pl.when(ph == 0)
    def _():
        hn0 = s0[0] / jnp.maximum(d0[0], 1.0)
        hn1 = s1[0] / jnp.maximum(d1[0], 1.0)
        o0 = (jnp.dot(h[...], ws0[...], preferred_element_type=jnp.float32)
              + jnp.dot(hn0, wn0[...], preferred_element_type=jnp.float32)
              + b0[...])
        o1 = (jnp.dot(h[...], ws1[...], preferred_element_type=jnp.float32)
              + jnp.dot(hn1, wn1[...], preferred_element_type=jnp.float32)
              + b1[...])
        if relu:
            o0 = jnp.maximum(o0, 0.0)
            o1 = jnp.maximum(o1, 0.0)
        a = o0 + o1
        acc_s[pl.ds(j * _BLK, _BLK), :] = a
        cs = jnp.sum(a, axis=0, keepdims=True)
        cq = jnp.sum(a * a, axis=0, keepdims=True)

        @pl.when(j == 0)
        def _():
            sums[0:1, :] = cs
            sums[1:2, :] = cq

        @pl.when(j > 0)
        def _():
            sums[0:1, :] = sums[0:1, :] + cs
            sums[1:2, :] = sums[1:2, :] + cq

    @pl.when(ph == 1)
    def _():
        mean = sums[0:1, :] * (1.0 / _N)
        var = sums[1:2, :] * (1.0 / _N) - mean * mean
        a = acc_s[pl.ds(j * _BLK, _BLK), :]
        out[...] = (a - mean) * lax.rsqrt(var + 1e-5) * g[...] + bt[...]


def _make_tc(relu):
    blk = lambda p, j: (j, 0)
    sblk = lambda p, j: (0, j, 0)
    dblk = lambda p, j: (0, j, 0)
    whole = lambda p, j: (0, 0)
    in_specs = [
        pl.BlockSpec((_BLK, _D), blk),
        pl.BlockSpec((1, _BLK, _D), sblk),
        pl.BlockSpec((1, _BLK, _D), lambda p, j: (1, j, 0)),
        pl.BlockSpec((1, _BLK, 1), dblk),
        pl.BlockSpec((1, _BLK, 1), lambda p, j: (1, j, 0)),
        pl.BlockSpec((_D, _D), whole),
        pl.BlockSpec((_D, _D), whole),
        pl.BlockSpec((1, _D), whole),
        pl.BlockSpec((_D, _D), whole),
        pl.BlockSpec((_D, _D), whole),
        pl.BlockSpec((1, _D), whole),
        pl.BlockSpec((1, _D), whole),
        pl.BlockSpec((1, _D), whole),
    ]
    return pl.pallas_call(
        functools.partial(_tc_body, relu),
        grid=(2, _NB),
        in_specs=in_specs,
        out_specs=pl.BlockSpec((_BLK, _D), blk),
        out_shape=jax.ShapeDtypeStruct((_N, _D), jnp.float32),
        scratch_shapes=[
            pltpu.VMEM((_N, _D), jnp.float32),
            pltpu.VMEM((8, _D), jnp.float32),
        ],
    )


def _edge_layout(ei):
    s = ei[0].reshape(_NTILE, _PT)
    d = ei[1].reshape(_NTILE, _PT)
    s = jnp.pad(s, ((0, 0), (0, _PTP - _PT)))
    d = jnp.pad(d, ((0, 0), (0, _PTP - _PT)), constant_values=_N)
    return s.reshape(_NTILE, _NCHP, _CH), d.reshape(_NTILE, _NCHP, _CH)


def kernel(x, edge_index_0, edge_index_1,
           W_self_0_0, W_neigh_0_0, b_0_0,
           W_self_0_1, W_neigh_0_1, b_0_1,
           gamma_0, beta_0,
           W_self_1_0, W_neigh_1_0, b_1_0,
           W_self_1_1, W_neigh_1_1, b_1_1,
           gamma_1, beta_1,
           W_self_2_0, W_neigh_2_0, b_2_0,
           W_self_2_1, W_neigh_2_1, b_2_1,
           gamma_2, beta_2):
    s0, d0 = _edge_layout(edge_index_0)
    s1, d1 = _edge_layout(edge_index_1)
    z2d = jnp.zeros((_ZLAST, _D), jnp.float32)
    z1d = jnp.zeros((_DCH,), jnp.float32)

    sc_first = _make_sc(True)
    sc_rest = _make_sc(False)
    tc_mid = _make_tc(True)
    tc_last = _make_tc(False)

    edge_args = (s0, d0, s1, d1, z2d, z1d)

    layer_ws = [
        (W_self_0_0, W_neigh_0_0, b_0_0, W_self_0_1, W_neigh_0_1, b_0_1,
         gamma_0, beta_0),
        (W_self_1_0, W_neigh_1_0, b_1_0, W_self_1_1, W_neigh_1_1, b_1_1,
         gamma_1, beta_1),
        (W_self_2_0, W_neigh_2_0, b_2_0, W_self_2_1, W_neigh_2_1, b_2_1,
         gamma_2, beta_2),
    ]

    h = x
    deg = None
    for l in range(3):
        if l == 0:
            S, g0, g1 = sc_first(h, *edge_args)
            deg = jnp.stack([g0, g1]).reshape(2, _N, 1)
        else:
            (S,) = sc_rest(h, *edge_args)
        ws0, wn0, b0, ws1, wn1, b1, g, bt = layer_ws[l]
        tc = tc_mid if l < 2 else tc_last
        h = tc(h, S, S, deg, deg,
               ws0, wn0, b0.reshape(1, _D), ws1, wn1, b1.reshape(1, _D),
               g.reshape(1, _D), bt.reshape(1, _D))
    return h
